# Initial kernel scaffold; baseline (speedup 1.0000x reference)
import jax
import jax.numpy as jnp
from jax.experimental import pallas as pl


def kernel(x, perm, L1_rows, L1_cols, L1_vals, L2_rows, L2_cols, L2_vals, W1, b1, W2, b2, Wfc1, bfc1):
    raise NotImplementedError("write your pallas kernel here")


# speedup vs baseline: 4.4974x; 4.4974x over previous
import functools

import jax
import jax.numpy as jnp
from jax import lax
from jax.experimental import pallas as pl
from jax.experimental.pallas import tpu as pltpu
from jax.experimental.pallas import tpu_sc as plsc

B = 4
C = 128
BC = B * C
IN_V = 16000
VP = 16384
DEG = 32
V2 = VP // 4
NC = 2
NS = 16
NW = NC * NS
NVREG = BC // 16


def _worker_id():
    return lax.axis_index("s") * NC + lax.axis_index("c")


def _sc_perm_gather(src, perm):
    V = perm.shape[0]
    rpw = V // NW
    CH = 64

    mesh = plsc.VectorSubcoreMesh(core_axis_name="c", subcore_axis_name="s")

    @functools.partial(
        pl.kernel,
        mesh=mesh,
        out_type=jax.ShapeDtypeStruct((V, BC), jnp.float32),
        scratch_types=[
            pltpu.VMEM((CH,), jnp.int32),
            pltpu.VMEM((CH, BC), jnp.float32),
            pltpu.SemaphoreType.DMA,
        ],
    )
    def k(src_hbm, perm_hbm, out_hbm, idx_v, rows_v, sem):
        wid = _worker_id()

        def chunk(t, carry):
            base = wid * rpw + t * CH
            pltpu.sync_copy(perm_hbm.at[pl.ds(base, CH)], idx_v)
            pltpu.async_copy(src_hbm.at[idx_v], rows_v, sem).wait()
            pltpu.sync_copy(rows_v, out_hbm.at[pl.ds(base, CH)])
            return carry

        lax.fori_loop(0, rpw // CH, chunk, 0)

    return k(src, perm)


def _sc_spmm(m, cols, vals):
    V = m.shape[0]
    rpw = V // NW

    mesh = plsc.VectorSubcoreMesh(core_axis_name="c", subcore_axis_name="s")

    @functools.partial(
        pl.kernel,
        mesh=mesh,
        out_type=jax.ShapeDtypeStruct((V, BC), jnp.float32),
        scratch_types=[
            pltpu.VMEM((DEG,), jnp.int32),
            pltpu.VMEM((DEG,), jnp.float32),
            pltpu.VMEM((DEG, BC), jnp.float32),
            pltpu.VMEM((BC,), jnp.float32),
            pltpu.SemaphoreType.DMA,
        ],
    )
    def k(m_hbm, cols_hbm, vals_hbm, out_hbm, idx_v, vals_v, rows_v, stage_v, sem):
        wid = _worker_id()

        def row_body(r, carry):
            row = wid * rpw + r
            e0 = row * DEG
            pltpu.sync_copy(cols_hbm.at[pl.ds(e0, DEG)], idx_v)
            pltpu.sync_copy(vals_hbm.at[pl.ds(e0, DEG)], vals_v)
            pltpu.async_copy(m_hbm.at[idx_v], rows_v, sem).wait()
            vlo = vals_v[pl.ds(0, 16)]
            vhi = vals_v[pl.ds(16, 16)]

            def edge(j, acc):
                jm = jnp.full((16,), j & 15, jnp.int32)
                src = jnp.where(j < 16, vlo, vhi)
                vj = src.at[jm].get(mode="promise_in_bounds")
                return tuple(
                    acc[c] + vj * rows_v[j, pl.ds(c * 16, 16)]
                    for c in range(NVREG)
                )

            acc0 = tuple(jnp.zeros((16,), jnp.float32) for _ in range(NVREG))
            acc = lax.fori_loop(0, DEG, edge, acc0)
            for c in range(NVREG):
                stage_v[pl.ds(c * 16, 16)] = acc[c]
            pltpu.sync_copy(stage_v, out_hbm.at[row])
            return carry

        lax.fori_loop(0, rpw, row_body, 0)

    return k(m, cols, vals)


def _tc_conv(xs0, xs1, xst, w, bias, vt):
    V = xs0.shape[0]
    F = w.shape[0]

    def body(x0_ref, x1_ref, xt_ref, w_ref, b_ref, o_ref):
        for b in range(B):
            sl = pl.ds(b * C, C)
            X = jnp.concatenate(
                [x0_ref[:, sl], x1_ref[:, sl], xt_ref[:, sl]], axis=1)
            Y = lax.dot_general(X, w_ref[...], (((1,), (1,)), ((), ())),
                                preferred_element_type=jnp.float32)
            Z = jnp.maximum(Y + b_ref[...], 0.0)
            o_ref[:, b, :] = Z.reshape(vt // 4, 4, F).max(axis=1)

    grid = (V // vt,)
    xspec = pl.BlockSpec((vt, BC), lambda i: (i, 0))
    return pl.pallas_call(
        body,
        grid=grid,
        in_specs=[
            xspec, xspec, xspec,
            pl.BlockSpec((F, 3 * C), lambda i: (0, 0)),
            pl.BlockSpec((1, F), lambda i: (0, 0)),
        ],
        out_specs=pl.BlockSpec((vt // 4, B, F), lambda i: (i, 0, 0)),
        out_shape=jax.ShapeDtypeStruct((V // 4, B, F), jnp.float32),
    )(xs0, xs1, xst, w, bias)


def _tc_fc(act, wfc, bias, kt):
    Bx, K = act.shape
    F = wfc.shape[0]

    def body(a_ref, w_ref, b_ref, o_ref):
        @pl.when(pl.program_id(0) == 0)
        def _init():
            o_ref[...] = jnp.broadcast_to(b_ref[...], o_ref.shape)

        o_ref[...] += lax.dot_general(
            a_ref[...], w_ref[...], (((1,), (1,)), ((), ())),
            preferred_element_type=jnp.float32)

    return pl.pallas_call(
        body,
        grid=(K // kt,),
        in_specs=[
            pl.BlockSpec((Bx, kt), lambda k: (0, k)),
            pl.BlockSpec((F, kt), lambda k: (0, k)),
            pl.BlockSpec((1, F), lambda k: (0, 0)),
        ],
        out_specs=pl.BlockSpec((Bx, F), lambda k: (0, 0)),
        out_shape=jax.ShapeDtypeStruct((Bx, F), jnp.float32),
    )(act, wfc, bias)


def _prep_w(W, scale):
    F = W.shape[0]
    Wk = W.reshape(F, C, 3)
    W0 = Wk[:, :, 0] - Wk[:, :, 2]
    Wa = Wk[:, :, 1]
    Wb = 2.0 * Wk[:, :, 2]
    return jnp.concatenate([W0, Wa, Wb], axis=1) * scale


def kernel(x, perm, L1_rows, L1_cols, L1_vals, L2_rows, L2_cols, L2_vals,
           W1, b1, W2, b2, Wfc1, bfc1):
    del L1_rows, L2_rows
    s = 1.0 / jnp.sqrt(jnp.float32(1.0 + 1e-5))

    xr = jnp.transpose(x, (2, 0, 1)).reshape(IN_V, BC)
    xr = jnp.pad(xr, ((0, VP - IN_V), (0, 0)))

    x0 = _sc_perm_gather(xr, perm)
    x1 = _sc_spmm(x0, L1_cols, L1_vals)
    t1 = _sc_spmm(x1, L1_cols, L1_vals)
    w1e = _prep_w(W1, s)
    p1 = _tc_conv(x0, x1, t1, w1e, b1.reshape(1, -1), vt=1024)
    m2 = p1.reshape(V2, BC)

    x1b = _sc_spmm(m2, L2_cols, L2_vals)
    t2 = _sc_spmm(x1b, L2_cols, L2_vals)
    w2e = _prep_w(W2, 1.0)
    p2 = _tc_conv(m2, x1b, t2, w2e, b2.reshape(1, -1), vt=1024)

    act = jnp.transpose(p2, (1, 2, 0)).reshape(B, -1)
    return _tc_fc(act, Wfc1, bfc1.reshape(1, -1), kt=8192)


# speedup vs baseline: 8.1571x; 1.8137x over previous
import functools

import jax
import jax.numpy as jnp
from jax import lax
from jax.experimental import pallas as pl
from jax.experimental.pallas import tpu as pltpu
from jax.experimental.pallas import tpu_sc as plsc

B = 4
C = 128
BC = B * C
IN_V = 16000
VP = 16384
DEG = 32
V2 = VP // 4
NC = 2
NS = 16
NW = NC * NS
NVREG = BC // 16


def _worker_id():
    return lax.axis_index("s") * NC + lax.axis_index("c")


def _sc_perm_gather(src, perm):
    V = perm.shape[0]
    rpw = V // NW
    CH = 64

    mesh = plsc.VectorSubcoreMesh(core_axis_name="c", subcore_axis_name="s")

    @functools.partial(
        pl.kernel,
        mesh=mesh,
        out_type=jax.ShapeDtypeStruct((V, BC), jnp.float32),
        scratch_types=[
            pltpu.VMEM((CH,), jnp.int32),
            pltpu.VMEM((CH, BC), jnp.float32),
            pltpu.SemaphoreType.DMA,
        ],
    )
    def k(src_hbm, perm_hbm, out_hbm, idx_v, rows_v, sem):
        wid = _worker_id()

        def chunk(t, carry):
            base = wid * rpw + t * CH
            pltpu.sync_copy(perm_hbm.at[pl.ds(base, CH)], idx_v)
            pltpu.async_copy(src_hbm.at[idx_v], rows_v, sem).wait()
            pltpu.sync_copy(rows_v, out_hbm.at[pl.ds(base, CH)])
            return carry

        lax.fori_loop(0, rpw // CH, chunk, 0)

    return k(src, perm)


def _sc_spmm(m, cols, vals):
    V = m.shape[0]
    rpw = V // NW
    cols2 = cols.reshape(V, DEG)
    vals2 = vals.reshape(V, DEG)

    mesh = plsc.VectorSubcoreMesh(core_axis_name="c", subcore_axis_name="s")

    CHI = 64
    H = 16

    @functools.partial(
        pl.kernel,
        mesh=mesh,
        out_type=jax.ShapeDtypeStruct((V, BC), jnp.float32),
        scratch_types=[
            pltpu.VMEM((CHI, DEG), jnp.int32),
            pltpu.VMEM((CHI, DEG), jnp.float32),
            pltpu.VMEM((2, H, BC), jnp.float32),
            pltpu.VMEM((BC,), jnp.float32),
            pltpu.SemaphoreType.DMA,
            pltpu.SemaphoreType.DMA,
        ],
    )
    def k(m_hbm, cols_hbm, vals_hbm, out_hbm, idxc, valsc, rows2, stage_v, sem0, sem1):
        wid = _worker_id()
        r0g = wid * rpw

        def compute_half(q, p, buf, acc):
            vsrc = valsc[q, pl.ds(p * H, 16)]

            def edge(j, a):
                vj = vsrc.at[jnp.full((16,), j, jnp.int32)].get(
                    mode="promise_in_bounds")
                return tuple(
                    a[c] + vj * rows2[buf, j, pl.ds(c * 16, 16)]
                    for c in range(NVREG)
                )

            return lax.fori_loop(0, H, edge, acc)

        def blk_body(blk, carry):
            row0 = r0g + blk * CHI
            pltpu.sync_copy(cols_hbm.at[pl.ds(row0, CHI)], idxc)
            pltpu.sync_copy(vals_hbm.at[pl.ds(row0, CHI)], valsc)
            pltpu.async_copy(m_hbm.at[idxc.at[0, pl.ds(0, H)]], rows2.at[0], sem0)

            def row_body(q, c2):
                pltpu.async_copy(
                    m_hbm.at[idxc.at[q, pl.ds(H, H)]], rows2.at[1], sem1)
                pltpu.make_async_copy(
                    m_hbm.at[idxc.at[q, pl.ds(0, H)]], rows2.at[0], sem0).wait()
                acc0 = tuple(jnp.zeros((16,), jnp.float32) for _ in range(NVREG))
                acc = compute_half(q, 0, 0, acc0)

                @pl.when(q + 1 < CHI)
                def _prefetch():
                    pltpu.async_copy(
                        m_hbm.at[idxc.at[q + 1, pl.ds(0, H)]], rows2.at[0], sem0)

                pltpu.make_async_copy(
                    m_hbm.at[idxc.at[q, pl.ds(H, H)]], rows2.at[1], sem1).wait()
                acc = compute_half(q, 1, 1, acc)
                for c in range(NVREG):
                    stage_v[pl.ds(c * 16, 16)] = acc[c]
                pltpu.sync_copy(stage_v, out_hbm.at[row0 + q])
                return c2

            lax.fori_loop(0, CHI, row_body, 0)
            return carry

        lax.fori_loop(0, rpw // CHI, blk_body, 0)

    return k(m, cols2, vals2)


def _tc_conv(xs0, xs1, xst, w, bias, vt):
    V = xs0.shape[0]
    F = w.shape[0]

    def body(x0_ref, x1_ref, xt_ref, w_ref, b_ref, o_ref):
        for b in range(B):
            sl = pl.ds(b * C, C)
            X = jnp.concatenate(
                [x0_ref[:, sl], x1_ref[:, sl], xt_ref[:, sl]], axis=1)
            Y = lax.dot_general(X, w_ref[...], (((1,), (1,)), ((), ())),
                                preferred_element_type=jnp.float32)
            Z = jnp.maximum(Y + b_ref[...], 0.0)
            o_ref[:, b, :] = Z.reshape(vt // 4, 4, F).max(axis=1)

    grid = (V // vt,)
    xspec = pl.BlockSpec((vt, BC), lambda i: (i, 0))
    return pl.pallas_call(
        body,
        grid=grid,
        in_specs=[
            xspec, xspec, xspec,
            pl.BlockSpec((F, 3 * C), lambda i: (0, 0)),
            pl.BlockSpec((1, F), lambda i: (0, 0)),
        ],
        out_specs=pl.BlockSpec((vt // 4, B, F), lambda i: (i, 0, 0)),
        out_shape=jax.ShapeDtypeStruct((V // 4, B, F), jnp.float32),
    )(xs0, xs1, xst, w, bias)


def _tc_fc(act, wfc, bias, kt):
    Bx, K = act.shape
    F = wfc.shape[0]

    def body(a_ref, w_ref, b_ref, o_ref):
        @pl.when(pl.program_id(0) == 0)
        def _init():
            o_ref[...] = jnp.broadcast_to(b_ref[...], o_ref.shape)

        o_ref[...] += lax.dot_general(
            a_ref[...], w_ref[...], (((1,), (1,)), ((), ())),
            preferred_element_type=jnp.float32)

    return pl.pallas_call(
        body,
        grid=(K // kt,),
        in_specs=[
            pl.BlockSpec((Bx, kt), lambda k: (0, k)),
            pl.BlockSpec((F, kt), lambda k: (0, k)),
            pl.BlockSpec((1, F), lambda k: (0, 0)),
        ],
        out_specs=pl.BlockSpec((Bx, F), lambda k: (0, 0)),
        out_shape=jax.ShapeDtypeStruct((Bx, F), jnp.float32),
    )(act, wfc, bias)


def _prep_w(W, scale):
    F = W.shape[0]
    Wk = W.reshape(F, C, 3)
    W0 = Wk[:, :, 0] - Wk[:, :, 2]
    Wa = Wk[:, :, 1]
    Wb = 2.0 * Wk[:, :, 2]
    return jnp.concatenate([W0, Wa, Wb], axis=1) * scale


def kernel(x, perm, L1_rows, L1_cols, L1_vals, L2_rows, L2_cols, L2_vals,
           W1, b1, W2, b2, Wfc1, bfc1):
    del L1_rows, L2_rows
    s = 1.0 / jnp.sqrt(jnp.float32(1.0 + 1e-5))

    xr = jnp.transpose(x, (2, 0, 1)).reshape(IN_V, BC)
    xr = jnp.pad(xr, ((0, VP - IN_V), (0, 0)))

    x0 = _sc_perm_gather(xr, perm)
    x1 = _sc_spmm(x0, L1_cols, L1_vals)
    t1 = _sc_spmm(x1, L1_cols, L1_vals)
    w1e = _prep_w(W1, s)
    p1 = _tc_conv(x0, x1, t1, w1e, b1.reshape(1, -1), vt=1024)
    m2 = p1.reshape(V2, BC)

    x1b = _sc_spmm(m2, L2_cols, L2_vals)
    t2 = _sc_spmm(x1b, L2_cols, L2_vals)
    w2e = _prep_w(W2, 1.0)
    p2 = _tc_conv(m2, x1b, t2, w2e, b2.reshape(1, -1), vt=1024)

    act = jnp.transpose(p2, (1, 2, 0)).reshape(B, -1)
    return _tc_fc(act, Wfc1, bfc1.reshape(1, -1), kt=8192)
